# Initial kernel scaffold; baseline (speedup 1.0000x reference)
import jax
import jax.numpy as jnp
from jax.experimental import pallas as pl


def kernel(x, W):
    raise NotImplementedError("write your pallas kernel here")


# speedup vs baseline: 1.0065x; 1.0065x over previous
import functools

import jax
import jax.numpy as jnp
from jax import lax
from jax.experimental import pallas as pl
from jax.experimental.pallas import tpu as pltpu
from jax.experimental.pallas import tpu_sc as plsc

_B, _C, _H, _W = 8, 256, 32, 32
_HW = _H * _W
_N = _B * _HW
_K = 8192

_TK = 512
_KT = _K // _TK

_NC, _NS = 2, 16
_NW = _NC * _NS
_RPW = _N // _NW


def _argmin_body(x_ref, w_ref, idx_ref, loss_ref, bestv, besti):
    b = pl.program_id(0)
    j = pl.program_id(1)
    xb = x_ref[0]
    wt = w_ref[...]
    m = jnp.dot(wt, xb, preferred_element_type=jnp.float32)
    wsq = jnp.sum(wt * wt, axis=1, keepdims=True)
    xsq = jnp.sum(xb * xb, axis=0, keepdims=True)
    d = (xsq + wsq) - 2.0 * m
    dmin = jnp.min(d, axis=0, keepdims=True)
    ii = lax.broadcasted_iota(jnp.int32, d.shape, 0)
    li = jnp.min(jnp.where(d == dmin, ii, _K), axis=0, keepdims=True) + j * _TK

    @pl.when(j == 0)
    def _init():
        bestv[...] = dmin
        besti[...] = li

    @pl.when(j > 0)
    def _update():
        pv = bestv[...]
        upd = dmin < pv
        bestv[...] = jnp.where(upd, dmin, pv)
        besti[...] = jnp.where(upd, li, besti[...])

    @pl.when(j == _KT - 1)
    def _finalize():
        idx_ref[0] = besti[...]
        s = jnp.sum(bestv[...])

        @pl.when(b == 0)
        def _():
            loss_ref[0, 0] = s

        @pl.when(b > 0)
        def _():
            loss_ref[0, 0] += s


_argmin_call = pl.pallas_call(
    _argmin_body,
    grid=(_B, _KT),
    in_specs=[
        pl.BlockSpec((1, _C, _HW), lambda b, j: (b, 0, 0)),
        pl.BlockSpec((_TK, _C), lambda b, j: (j, 0)),
    ],
    out_specs=[
        pl.BlockSpec((1, 1, _HW), lambda b, j: (b, 0, 0)),
        pl.BlockSpec((1, 1), lambda b, j: (0, 0), memory_space=pltpu.SMEM),
    ],
    out_shape=[
        jax.ShapeDtypeStruct((_B, 1, _HW), jnp.int32),
        jax.ShapeDtypeStruct((1, 1), jnp.float32),
    ],
    scratch_shapes=[
        pltpu.VMEM((1, _HW), jnp.float32),
        pltpu.VMEM((1, _HW), jnp.int32),
    ],
    compiler_params=pltpu.CompilerParams(
        dimension_semantics=("arbitrary", "arbitrary"),
    ),
)


@functools.cache
def _gather_rows_call():
    @functools.partial(
        pl.kernel,
        out_type=jax.ShapeDtypeStruct((_N, _C), jnp.float32),
        mesh=plsc.VectorSubcoreMesh(core_axis_name="c", subcore_axis_name="s"),
        scratch_types=[
            pltpu.VMEM((_RPW,), jnp.int32),
            pltpu.VMEM((_RPW, _C), jnp.float32),
            pltpu.SemaphoreType.DMA,
        ],
    )
    def _gather_rows(w_hbm, idx_hbm, z_hbm, idx_v, rows_v, sem):
        wid = lax.axis_index("s") * _NC + lax.axis_index("c")
        base = wid * _RPW
        pltpu.sync_copy(idx_hbm.at[pl.ds(base, _RPW)], idx_v)
        pltpu.async_copy(w_hbm.at[idx_v], rows_v, sem).wait()
        pltpu.sync_copy(rows_v, z_hbm.at[pl.ds(base, _RPW)])

    return _gather_rows


def kernel(x, W):
    B, C, H, Wd = x.shape
    xr = x.reshape(B, C, H * Wd)
    midx, loss_sum = _argmin_call(xr, W)
    idx_flat = midx.reshape(B * H * Wd)
    z = _gather_rows_call()(W, idx_flat)
    z_out = jnp.transpose(z.reshape(B, H, Wd, C), (0, 3, 1, 2))
    sequence = midx.reshape(B, H, Wd)
    loss = loss_sum[0, 0] / (B * H * Wd * C)
    return (z_out, sequence, loss, loss)
